# Initial kernel scaffold; baseline (speedup 1.0000x reference)
import jax
import jax.numpy as jnp
from jax.experimental import pallas as pl


def kernel(hypo_scores, next_token_probs, beam_width):
    raise NotImplementedError("write your pallas kernel here")


# speedup vs baseline: 87.4199x; 87.4199x over previous
import functools

import jax
import jax.numpy as jnp
from jax import lax
from jax.experimental import pallas as pl
from jax.experimental.pallas import tpu as pltpu
from jax.experimental.pallas import tpu_sc as plsc

R = 128
V = 100000
N = R * V
NC, NS = 2, 16
NW = NC * NS
RPW = R // NW
EPW = N // NW
SUB = 2000
SPR = V // SUB
SPW = EPW // SUB
CH = 20000
CPW = EPW // CH
K = 32
CAP = 512
IMAX = 2147483647

_mesh = plsc.VectorSubcoreMesh(core_axis_name="c", subcore_axis_name="s")


def _wid():
    return lax.axis_index("c") * NS + lax.axis_index("s")


def _neg16():
    return jnp.full((16,), -jnp.inf, jnp.float32)


def _vmax(v):
    out = plsc.sort_key_val(v, v, descending=True)
    k = out[0] if isinstance(out, (tuple, list)) else out
    return k[0]


def _vmin(v):
    out = plsc.sort_key_val(v, v, descending=False)
    k = out[0] if isinstance(out, (tuple, list)) else out
    return k[0]


def _popcnt(mask):
    return plsc.all_reduce_population_count(mask)[0]


@functools.partial(
    pl.kernel,
    mesh=_mesh,
    compiler_params=pltpu.CompilerParams(needs_layout_passes=False),
    out_type=[
        jax.ShapeDtypeStruct((NW * 16,), jnp.float32),
        jax.ShapeDtypeStruct((NW * SPW * 16,), jnp.float32),
    ],
    scratch_types=[
        pltpu.VMEM((CH,), jnp.float32),
        pltpu.VMEM((CH,), jnp.float32),
        pltpu.VMEM((SPW * 16,), jnp.float32),
        pltpu.VMEM((R + 16,), jnp.float32),
        pltpu.VMEM((16,), jnp.float32),
        pltpu.SemaphoreType.DMA,
        pltpu.SemaphoreType.DMA,
    ],
)
def _stage1(probs, hypo, wmax_o, trig_o, buf0, buf1, trigv, hypov, wmv, sem0, sem1):
    wid = _wid()
    base = wid * EPW
    pltpu.sync_copy(hypo, hypov.at[pl.ds(0, R)])
    pltpu.make_async_copy(probs.at[pl.ds(base, CH)], buf0, sem0).start()
    pltpu.make_async_copy(probs.at[pl.ds(base + CH, CH)], buf1, sem1).start()
    neg = _neg16()

    def chunk(c, bf, wm):
        h = hypov[pl.ds(RPW * wid + c // (CPW // RPW), 16)][0]
        hv = jnp.full((16,), h, jnp.float32)

        def sub(s, wm2):
            def jb(j, m):
                o = s * SUB + j * 400
                for u in range(25):
                    m = jnp.maximum(m, bf[pl.ds(o + u * 16, 16)])
                return m

            m = lax.fori_loop(0, 5, jb, neg)
            sm = m + hv
            trigv[pl.ds((c * (CH // SUB) + s) * 16, 16)] = sm
            return jnp.maximum(wm2, sm)

        return lax.fori_loop(0, CH // SUB, sub, wm)

    def outer(g, wm):
        c0 = 2 * g
        pltpu.make_async_copy(probs.at[pl.ds(base, CH)], buf0, sem0).wait()
        wm = chunk(c0, buf0, wm)

        @pl.when(c0 + 2 < CPW)
        def _():
            pltpu.make_async_copy(
                probs.at[pl.ds(base + (c0 + 2) * CH, CH)], buf0, sem0).start()

        pltpu.make_async_copy(probs.at[pl.ds(base, CH)], buf1, sem1).wait()
        wm = chunk(c0 + 1, buf1, wm)

        @pl.when(c0 + 3 < CPW)
        def _():
            pltpu.make_async_copy(
                probs.at[pl.ds(base + (c0 + 3) * CH, CH)], buf1, sem1).start()

        return wm

    wm = lax.fori_loop(0, CPW // 2, outer, neg)
    wmv[...] = wm
    pltpu.sync_copy(wmv, wmax_o.at[pl.ds(wid * 16, 16)])
    pltpu.sync_copy(trigv, trig_o.at[pl.ds(wid * SPW * 16, SPW * 16)])


@functools.partial(
    pl.kernel,
    mesh=_mesh,
    compiler_params=pltpu.CompilerParams(needs_layout_passes=False),
    out_type=[
        jax.ShapeDtypeStruct((NW * 16,), jnp.int32),
        jax.ShapeDtypeStruct((NW * CAP,), jnp.float32),
        jax.ShapeDtypeStruct((NW * CAP,), jnp.int32),
    ],
    scratch_types=[
        pltpu.VMEM((NW * 16,), jnp.float32),
        pltpu.VMEM((SPW * 16,), jnp.float32),
        pltpu.VMEM((R + 16,), jnp.float32),
        pltpu.VMEM((SUB,), jnp.float32),
        pltpu.VMEM((CAP,), jnp.float32),
        pltpu.VMEM((CAP,), jnp.int32),
        pltpu.VMEM((16,), jnp.int32),
    ],
)
def _stage2(probs, hypo, wmax_i, trig_i, cnt_o, cval_o, cidx_o,
            wmaxv, trigv, hypov, sbuf, cvalv, cidxv, cntv):
    wid = _wid()
    base = wid * EPW
    pltpu.sync_copy(hypo, hypov.at[pl.ds(0, R)])
    pltpu.sync_copy(wmax_i, wmaxv)
    pltpu.sync_copy(trig_i.at[pl.ds(wid * SPW * 16, SPW * 16)], trigv)
    neg = _neg16()

    def rnd(k, _):
        def mx(j, mm):
            return jnp.maximum(mm, wmaxv[pl.ds(j * 16, 16)])

        m = _vmax(lax.fori_loop(0, NW * 16 // 16, mx, neg))

        def mko(j, z):
            v = wmaxv[pl.ds(j * 16, 16)]
            wmaxv[pl.ds(j * 16, 16)] = jnp.where(
                v == m, jnp.float32(-jnp.inf), v)
            return z

        lax.fori_loop(0, NW * 16 // 16, mko, jnp.int32(0))
        return m

    T = lax.fori_loop(0, K, rnd, jnp.float32(0.0))
    Tv = jnp.full((16,), T, jnp.float32)
    iota = lax.iota(jnp.int32, 16)

    def ini(j, z):
        cvalv[pl.ds(j * 16, 16)] = neg
        return z

    lax.fori_loop(0, CAP // 16, ini, jnp.int32(0))

    def sub(s, cnt):
        h = hypov[pl.ds(RPW * wid + s // SPR, 16)][0]
        hv = jnp.full((16,), h, jnp.float32)
        tvec = trigv[pl.ds(s * 16, 16)]

        def do(cnt2):
            fb = base + s * SUB
            pltpu.sync_copy(probs.at[pl.ds(fb, SUB)], sbuf)

            def jb(j, c3):
                x = sbuf[pl.ds(j * 16, 16)]
                sv = hv + x
                mk = sv >= Tv
                pc = _popcnt(mk)

                def emit(c4):
                    c5 = jnp.minimum(c4, CAP - 16)
                    plsc.store_compressed(cvalv.at[pl.ds(c5, 16)], sv, mask=mk)
                    plsc.store_compressed(
                        cidxv.at[pl.ds(c5, 16)], iota + (fb + j * 16), mask=mk)
                    return c5 + pc

                return lax.cond(pc > 0, emit, lambda c4: c4, c3)

            return lax.fori_loop(0, SUB // 16, jb, cnt2)

        return lax.cond(_popcnt(tvec >= Tv) > 0, do, lambda c2: c2, cnt)

    cnt = lax.fori_loop(0, SPW, sub, jnp.int32(0))
    cntv[...] = jnp.full((16,), cnt, jnp.int32)
    pltpu.sync_copy(cntv, cnt_o.at[pl.ds(wid * 16, 16)])
    pltpu.sync_copy(cvalv, cval_o.at[pl.ds(wid * CAP, CAP)])
    pltpu.sync_copy(cidxv, cidx_o.at[pl.ds(wid * CAP, CAP)])


@functools.partial(
    pl.kernel,
    mesh=_mesh,
    compiler_params=pltpu.CompilerParams(needs_layout_passes=False),
    out_type=[
        jax.ShapeDtypeStruct((K,), jnp.float32),
        jax.ShapeDtypeStruct((K,), jnp.int32),
        jax.ShapeDtypeStruct((K,), jnp.int32),
    ],
    scratch_types=[
        pltpu.VMEM((NW * 16,), jnp.int32),
        pltpu.VMEM((NW * CAP,), jnp.float32),
        pltpu.VMEM((NW * CAP,), jnp.int32),
        pltpu.VMEM((NW * CAP + 16,), jnp.float32),
        pltpu.VMEM((NW * CAP + 16,), jnp.int32),
        pltpu.VMEM((K,), jnp.float32),
        pltpu.VMEM((K,), jnp.int32),
        pltpu.VMEM((K,), jnp.int32),
    ],
)
def _stage3(cnt_i, cval_i, cidx_i, s_o, h_o, t_o,
            cntv, cvalv, cidxv, mval, midx, osv, ohv, otv):
    wid = _wid()

    @pl.when(wid == 0)
    def _():
        pltpu.sync_copy(cnt_i, cntv)
        pltpu.sync_copy(cval_i, cvalv)
        pltpu.sync_copy(cidx_i, cidxv)
        neg = _neg16()
        iota = lax.iota(jnp.int32, 16)
        lane0 = iota == 0

        def cw(w, M):
            cnt = cntv[pl.ds(w * 16, 16)][0]
            nv = (cnt + 15) // 16

            def cj(j, z):
                mval[pl.ds(M + j * 16, 16)] = cvalv[pl.ds(w * CAP + j * 16, 16)]
                midx[pl.ds(M + j * 16, 16)] = cidxv[pl.ds(w * CAP + j * 16, 16)]
                return z

            lax.fori_loop(0, nv, cj, jnp.int32(0))
            return M + cnt

        M = lax.fori_loop(0, NW, cw, jnp.int32(0))
        nvv = (M + 15) // 16

        def rnd(k, z):
            def mx(j, mm):
                return jnp.maximum(mm, mval[pl.ds(j * 16, 16)])

            m = _vmax(lax.fori_loop(0, nvv, mx, neg))

            def mi(j, rm):
                v = mval[pl.ds(j * 16, 16)]
                iv = midx[pl.ds(j * 16, 16)]
                return jnp.minimum(rm, jnp.where(v == m, iv, IMAX))

            fi = _vmin(lax.fori_loop(0, nvv, mi, jnp.full((16,), IMAX, jnp.int32)))

            def mo(j, z2):
                v = mval[pl.ds(j * 16, 16)]
                iv = midx[pl.ds(j * 16, 16)]
                mval[pl.ds(j * 16, 16)] = jnp.where(
                    (v == m) & (iv == fi), jnp.float32(-jnp.inf), v)
                return z2

            lax.fori_loop(0, nvv, mo, jnp.int32(0))
            kv = jnp.full((16,), k, jnp.int32)
            hidx = fi // V
            plsc.store_scatter(osv, [kv], jnp.full((16,), m, jnp.float32), mask=lane0)
            plsc.store_scatter(ohv, [kv], jnp.full((16,), hidx, jnp.int32), mask=lane0)
            plsc.store_scatter(otv, [kv], jnp.full((16,), fi - hidx * V, jnp.int32),
                               mask=lane0)
            return z

        lax.fori_loop(0, K, rnd, jnp.int32(0))
        pltpu.sync_copy(osv, s_o)
        pltpu.sync_copy(ohv, h_o)
        pltpu.sync_copy(otv, t_o)


def kernel(hypo_scores, next_token_probs, beam_width):
    del beam_width
    flat = next_token_probs.reshape(-1)
    wmax, trig = _stage1(flat, hypo_scores)
    cnts, cval, cidx = _stage2(flat, hypo_scores, wmax, trig)
    scores, hidx, tok = _stage3(cnts, cval, cidx)
    return scores, hidx, tok


# speedup vs baseline: 246.2859x; 2.8173x over previous
import functools

import jax
import jax.numpy as jnp
from jax import lax
from jax.experimental import pallas as pl
from jax.experimental.pallas import tpu as pltpu
from jax.experimental.pallas import tpu_sc as plsc

R = 128
V = 100000
N = R * V
NC, NS = 2, 16
NW = NC * NS
BLK = 1024
NB = N // BLK
BPW = 391
CHB = 17
CHW = CHB * BLK
NCH = 24
K = 32
CAP = 512
IMAX = 2147483647

_mesh = plsc.VectorSubcoreMesh(core_axis_name="c", subcore_axis_name="s")
_params = pltpu.CompilerParams(needs_layout_passes=False)


def _wid():
    return lax.axis_index("c") * NS + lax.axis_index("s")


def _neg16():
    return jnp.full((16,), -jnp.inf, jnp.float32)


def _vmax(v):
    out = plsc.sort_key_val(v, v, descending=True)
    k = out[0] if isinstance(out, (tuple, list)) else out
    return k[0]


def _vmin(v):
    out = plsc.sort_key_val(v, v, descending=False)
    k = out[0] if isinstance(out, (tuple, list)) else out
    return k[0]


def _popcnt(mask):
    return plsc.all_reduce_population_count(mask)[0]


@functools.partial(
    pl.kernel,
    mesh=_mesh,
    compiler_params=_params,
    out_type=[
        jax.ShapeDtypeStruct((NW * 16,), jnp.int32),
        jax.ShapeDtypeStruct((NW * CAP,), jnp.float32),
        jax.ShapeDtypeStruct((NW * CAP,), jnp.int32),
    ],
    scratch_types=[
        pltpu.VMEM((CHW,), jnp.float32),
        pltpu.VMEM((CHW,), jnp.float32),
        pltpu.VMEM((BPW * 16,), jnp.float32),
        pltpu.VMEM((R,), jnp.float32),
        pltpu.VMEM((16,), jnp.float32),
        pltpu.VMEM_SHARED((NS * 16,), jnp.float32),
        pltpu.VMEM((NS * 16,), jnp.float32),
        pltpu.VMEM((BLK,), jnp.float32),
        pltpu.VMEM((CAP,), jnp.float32),
        pltpu.VMEM((CAP,), jnp.int32),
        pltpu.VMEM((16,), jnp.int32),
        pltpu.SemaphoreType.DMA,
        pltpu.SemaphoreType.DMA,
    ],
)
def _stage12(flat, hypo, cnt_o, cval_o, cidx_o, buf0, buf1, trigv, hypov, wmv,
             shared, smaxv, sbuf, cvalv, cidxv, cntv, sem0, sem1):
    wid = _wid()
    start_b = wid * BPW
    nb = jnp.minimum(BPW, NB - start_b)
    pltpu.sync_copy(hypo, hypov)
    hvs = [hypov[pl.ds(i * 16, 16)] for i in range(8)]
    neg = _neg16()

    def cstart(ch):
        return jnp.minimum(ch * CHB, nb - CHB)

    def issue(ch, bf, sem):
        pltpu.make_async_copy(
            flat.at[pl.ds((start_b + cstart(ch)) * BLK, CHW)], bf, sem).start()

    issue(0, buf0, sem0)
    issue(1, buf1, sem1)

    def chunk(ch, bf, wm):
        cs = cstart(ch)

        def blk(bi, wm2):
            o = bi * BLK
            ms = [neg] * 8
            for j in range(64):
                ms[j % 8] = jnp.maximum(ms[j % 8], bf[pl.ds(o + j * 16, 16)])
            t = ms[0] + hvs[0]
            for i in range(1, 8):
                t = jnp.maximum(t, ms[i] + hvs[i])
            trigv[pl.ds((cs + bi) * 16, 16)] = t
            return jnp.maximum(wm2, t)

        return lax.fori_loop(0, CHB, blk, wm)

    def outer(g, wm):
        c0 = 2 * g
        pltpu.make_async_copy(flat.at[pl.ds(0, CHW)], buf0, sem0).wait()
        wm = chunk(c0, buf0, wm)

        @pl.when(c0 + 2 < NCH)
        def _():
            issue(c0 + 2, buf0, sem0)

        pltpu.make_async_copy(flat.at[pl.ds(0, CHW)], buf1, sem1).wait()
        wm = chunk(c0 + 1, buf1, wm)

        @pl.when(c0 + 3 < NCH)
        def _():
            issue(c0 + 3, buf1, sem1)

        return wm

    wm = lax.fori_loop(0, NCH // 2, outer, neg)

    sid = lax.axis_index("s")
    wmv[...] = wm
    pltpu.sync_copy(wmv, shared.at[pl.ds(sid * 16, 16)])
    plsc.subcore_barrier()
    pltpu.sync_copy(shared, smaxv)

    def rnd(k, _):
        mm = neg
        for j in range(NS):
            mm = jnp.maximum(mm, smaxv[pl.ds(j * 16, 16)])
        m = _vmax(mm)
        ninf = jnp.float32(-jnp.inf)
        for j in range(NS):
            v = smaxv[pl.ds(j * 16, 16)]
            smaxv[pl.ds(j * 16, 16)] = jnp.where(v == m, ninf, v)
        return m

    T = lax.fori_loop(0, K, rnd, jnp.float32(0.0))
    Tv = jnp.full((16,), T, jnp.float32)
    iota = lax.iota(jnp.int32, 16)
    iotaV = iota * V

    def ini(j, z):
        cvalv[pl.ds(j * 16, 16)] = neg
        return z

    lax.fori_loop(0, CAP // 16, ini, jnp.int32(0))

    def fetch_emit(blk_abs, cnt3):
        pltpu.sync_copy(flat.at[pl.ds(blk_abs * BLK, BLK)], sbuf)
        c3 = cnt3
        for j in range(64):
            x = sbuf[pl.ds(j * 16, 16)]
            sv = x + hvs[j % 8]
            mk = sv >= Tv
            pc = _popcnt(mk)
            scal = (j % 8) * 16 * V + blk_abs * 8 + j // 8

            def emit(c4, sv=sv, mk=mk, scal=scal):
                c5 = jnp.minimum(c4, CAP - 16)
                plsc.store_compressed(cvalv.at[pl.ds(c5, 16)], sv, mask=mk)
                plsc.store_compressed(
                    cidxv.at[pl.ds(c5, 16)], iotaV + scal, mask=mk)
                return c5 + pc

            c3 = lax.cond(pc > 0, emit, lambda c4: c4, c3)
        return c3

    nbc = jnp.maximum(nb - 1, 0)

    def scan_blk(bi, cnt2):
        tvec = trigv[pl.ds(jnp.minimum(bi, nbc) * 16, 16)]
        hit = (_popcnt(tvec >= Tv) > 0) & (bi < nb)
        return lax.cond(
            hit, lambda c3: fetch_emit(start_b + bi, c3), lambda c3: c3, cnt2)

    def scan4(g, cnt):
        b0 = g * 4
        t = trigv[pl.ds(jnp.minimum(b0, nbc) * 16, 16)]
        for u in range(1, 4):
            t = jnp.maximum(t, trigv[pl.ds(jnp.minimum(b0 + u, nbc) * 16, 16)])

        def slow(cnt2):
            return lax.fori_loop(b0, b0 + 4, scan_blk, cnt2)

        return lax.cond(_popcnt(t >= Tv) > 0, slow, lambda c: c, cnt)

    cnt = lax.fori_loop(0, (BPW + 3) // 4, scan4, jnp.int32(0))
    cntv[...] = jnp.full((16,), cnt, jnp.int32)
    pltpu.sync_copy(cntv, cnt_o.at[pl.ds(wid * 16, 16)])
    pltpu.sync_copy(cvalv, cval_o.at[pl.ds(wid * CAP, CAP)])
    pltpu.sync_copy(cidxv, cidx_o.at[pl.ds(wid * CAP, CAP)])


@functools.partial(
    pl.kernel,
    mesh=_mesh,
    compiler_params=_params,
    out_type=[
        jax.ShapeDtypeStruct((K,), jnp.float32),
        jax.ShapeDtypeStruct((K,), jnp.int32),
        jax.ShapeDtypeStruct((K,), jnp.int32),
    ],
    scratch_types=[
        pltpu.VMEM((NW * 16,), jnp.int32),
        pltpu.VMEM((NW * CAP,), jnp.float32),
        pltpu.VMEM((NW * CAP,), jnp.int32),
        pltpu.VMEM((NW * CAP + 16,), jnp.float32),
        pltpu.VMEM((NW * CAP + 16,), jnp.int32),
        pltpu.VMEM((K,), jnp.float32),
        pltpu.VMEM((K,), jnp.int32),
        pltpu.VMEM((K,), jnp.int32),
    ],
)
def _stage3(cnt_i, cval_i, cidx_i, s_o, h_o, t_o,
            cntv, cvalv, cidxv, mval, midx, osv, ohv, otv):
    wid = _wid()

    @pl.when(wid == 0)
    def _():
        pltpu.sync_copy(cnt_i, cntv)
        pltpu.sync_copy(cval_i, cvalv)
        pltpu.sync_copy(cidx_i, cidxv)
        neg = _neg16()
        iota = lax.iota(jnp.int32, 16)
        lane0 = iota == 0

        def cw(w, M):
            cnt = cntv[pl.ds(w * 16, 16)][0]
            nv = (cnt + 15) // 16

            def cj(j, z):
                mval[pl.ds(M + j * 16, 16)] = cvalv[pl.ds(w * CAP + j * 16, 16)]
                midx[pl.ds(M + j * 16, 16)] = cidxv[pl.ds(w * CAP + j * 16, 16)]
                return z

            lax.fori_loop(0, nv, cj, jnp.int32(0))
            return M + cnt

        M = lax.fori_loop(0, NW, cw, jnp.int32(0))
        nvv = (M + 15) // 16

        def rnd(k, z):
            def mx(j, mm):
                return jnp.maximum(mm, mval[pl.ds(j * 16, 16)])

            m = _vmax(lax.fori_loop(0, nvv, mx, neg))

            def mi(j, rm):
                v = mval[pl.ds(j * 16, 16)]
                iv = midx[pl.ds(j * 16, 16)]
                return jnp.minimum(rm, jnp.where(v == m, iv, IMAX))

            fi = _vmin(lax.fori_loop(0, nvv, mi, jnp.full((16,), IMAX, jnp.int32)))

            def mo(j, z2):
                v = mval[pl.ds(j * 16, 16)]
                iv = midx[pl.ds(j * 16, 16)]
                mval[pl.ds(j * 16, 16)] = jnp.where(
                    (v == m) & (iv == fi), jnp.float32(-jnp.inf), v)
                return z2

            lax.fori_loop(0, nvv, mo, jnp.int32(0))
            kv = jnp.full((16,), k, jnp.int32)
            hidx = fi // V
            plsc.store_scatter(osv, [kv], jnp.full((16,), m, jnp.float32), mask=lane0)
            plsc.store_scatter(ohv, [kv], jnp.full((16,), hidx, jnp.int32), mask=lane0)
            plsc.store_scatter(otv, [kv], jnp.full((16,), fi - hidx * V, jnp.int32),
                               mask=lane0)
            return z

        lax.fori_loop(0, K, rnd, jnp.int32(0))
        pltpu.sync_copy(osv, s_o)
        pltpu.sync_copy(ohv, h_o)
        pltpu.sync_copy(otv, t_o)


def kernel(hypo_scores, next_token_probs, beam_width):
    del beam_width
    flat = next_token_probs.T.reshape(-1)
    cnts, cval, cidx = _stage12(flat, hypo_scores)
    scores, hidx, tok = _stage3(cnts, cval, cidx)
    return scores, hidx, tok


# speedup vs baseline: 251.0326x; 1.0193x over previous
import functools

import jax
import jax.numpy as jnp
from jax import lax
from jax.experimental import pallas as pl
from jax.experimental.pallas import tpu as pltpu
from jax.experimental.pallas import tpu_sc as plsc

R = 128
V = 100000
N = R * V
NC, NS = 2, 16
NW = NC * NS
BLK = 1024
NB = N // BLK
BPW = 391
CHB = 23
CHW = CHB * BLK
NCH = 18
K = 32
CAP = 512
IMAX = 2147483647

_mesh = plsc.VectorSubcoreMesh(core_axis_name="c", subcore_axis_name="s")
_params = pltpu.CompilerParams(needs_layout_passes=False)


def _wid():
    return lax.axis_index("c") * NS + lax.axis_index("s")


def _neg16():
    return jnp.full((16,), -jnp.inf, jnp.float32)


def _vmax(v):
    out = plsc.sort_key_val(v, v, descending=True)
    k = out[0] if isinstance(out, (tuple, list)) else out
    return k[0]


def _vmin(v):
    out = plsc.sort_key_val(v, v, descending=False)
    k = out[0] if isinstance(out, (tuple, list)) else out
    return k[0]


def _popcnt(mask):
    return plsc.all_reduce_population_count(mask)[0]


@functools.partial(
    pl.kernel,
    mesh=_mesh,
    compiler_params=_params,
    out_type=[
        jax.ShapeDtypeStruct((NW * 16,), jnp.int32),
        jax.ShapeDtypeStruct((NW * CAP,), jnp.float32),
        jax.ShapeDtypeStruct((NW * CAP,), jnp.int32),
    ],
    scratch_types=[
        pltpu.VMEM((CHW,), jnp.float32),
        pltpu.VMEM((CHW,), jnp.float32),
        pltpu.VMEM((BPW * 16,), jnp.float32),
        pltpu.VMEM((R,), jnp.float32),
        pltpu.VMEM((16,), jnp.float32),
        pltpu.VMEM_SHARED((NS * 16,), jnp.float32),
        pltpu.VMEM((NS * 16,), jnp.float32),
        pltpu.VMEM((BLK,), jnp.float32),
        pltpu.VMEM((CAP,), jnp.float32),
        pltpu.VMEM((CAP,), jnp.int32),
        pltpu.VMEM((16,), jnp.int32),
        pltpu.SemaphoreType.DMA,
        pltpu.SemaphoreType.DMA,
    ],
)
def _stage12(flat, hypo, cnt_o, cval_o, cidx_o, buf0, buf1, trigv, hypov, wmv,
             shared, smaxv, sbuf, cvalv, cidxv, cntv, sem0, sem1):
    wid = _wid()
    start_b = wid * BPW
    nb = jnp.minimum(BPW, NB - start_b)
    pltpu.sync_copy(hypo, hypov)
    hvs = [hypov[pl.ds(i * 16, 16)] for i in range(8)]
    neg = _neg16()

    def cstart(ch):
        return jnp.minimum(ch * CHB, nb - CHB)

    def issue(ch, bf, sem):
        pltpu.make_async_copy(
            flat.at[pl.ds((start_b + cstart(ch)) * BLK, CHW)], bf, sem).start()

    issue(0, buf0, sem0)
    issue(1, buf1, sem1)

    def chunk(ch, bf, wm):
        cs = cstart(ch)

        def blk(bi, wm2):
            o = bi * BLK
            ms = [neg] * 8
            for j in range(64):
                ms[j % 8] = jnp.maximum(ms[j % 8], bf[pl.ds(o + j * 16, 16)])
            t = ms[0] + hvs[0]
            for i in range(1, 8):
                t = jnp.maximum(t, ms[i] + hvs[i])
            trigv[pl.ds((cs + bi) * 16, 16)] = t
            return jnp.maximum(wm2, t)

        return lax.fori_loop(0, CHB, blk, wm)

    def outer(g, wm):
        c0 = 2 * g
        pltpu.make_async_copy(flat.at[pl.ds(0, CHW)], buf0, sem0).wait()
        wm = chunk(c0, buf0, wm)

        @pl.when(c0 + 2 < NCH)
        def _():
            issue(c0 + 2, buf0, sem0)

        pltpu.make_async_copy(flat.at[pl.ds(0, CHW)], buf1, sem1).wait()
        wm = chunk(c0 + 1, buf1, wm)

        @pl.when(c0 + 3 < NCH)
        def _():
            issue(c0 + 3, buf1, sem1)

        return wm

    wm = lax.fori_loop(0, NCH // 2, outer, neg)

    sid = lax.axis_index("s")
    wmv[...] = wm
    pltpu.sync_copy(wmv, shared.at[pl.ds(sid * 16, 16)])
    plsc.subcore_barrier()
    pltpu.sync_copy(shared, smaxv)

    def rnd(k, _):
        mm = neg
        for j in range(NS):
            mm = jnp.maximum(mm, smaxv[pl.ds(j * 16, 16)])
        m = _vmax(mm)
        ninf = jnp.float32(-jnp.inf)
        for j in range(NS):
            v = smaxv[pl.ds(j * 16, 16)]
            smaxv[pl.ds(j * 16, 16)] = jnp.where(v == m, ninf, v)
        return m

    T = lax.fori_loop(0, K, rnd, jnp.float32(0.0))
    Tv = jnp.full((16,), T, jnp.float32)
    iota = lax.iota(jnp.int32, 16)
    iotaV = iota * V

    def ini(j, z):
        cvalv[pl.ds(j * 16, 16)] = neg
        return z

    lax.fori_loop(0, CAP // 16, ini, jnp.int32(0))

    def fetch_emit(blk_abs, cnt3):
        pltpu.sync_copy(flat.at[pl.ds(blk_abs * BLK, BLK)], sbuf)
        c3 = cnt3
        for j in range(64):
            x = sbuf[pl.ds(j * 16, 16)]
            sv = x + hvs[j % 8]
            mk = sv >= Tv
            pc = _popcnt(mk)
            scal = (j % 8) * 16 * V + blk_abs * 8 + j // 8

            def emit(c4, sv=sv, mk=mk, scal=scal):
                c5 = jnp.minimum(c4, CAP - 16)
                plsc.store_compressed(cvalv.at[pl.ds(c5, 16)], sv, mask=mk)
                plsc.store_compressed(
                    cidxv.at[pl.ds(c5, 16)], iotaV + scal, mask=mk)
                return c5 + pc

            c3 = lax.cond(pc > 0, emit, lambda c4: c4, c3)
        return c3

    nbc = jnp.maximum(nb - 1, 0)

    def scan_blk(bi, cnt2):
        tvec = trigv[pl.ds(jnp.minimum(bi, nbc) * 16, 16)]
        hit = (_popcnt(tvec >= Tv) > 0) & (bi < nb)
        return lax.cond(
            hit, lambda c3: fetch_emit(start_b + bi, c3), lambda c3: c3, cnt2)

    def scan4(g, cnt):
        b0 = g * 4
        t = trigv[pl.ds(jnp.minimum(b0, nbc) * 16, 16)]
        for u in range(1, 4):
            t = jnp.maximum(t, trigv[pl.ds(jnp.minimum(b0 + u, nbc) * 16, 16)])

        def slow(cnt2):
            return lax.fori_loop(b0, b0 + 4, scan_blk, cnt2)

        return lax.cond(_popcnt(t >= Tv) > 0, slow, lambda c: c, cnt)

    cnt = lax.fori_loop(0, (BPW + 3) // 4, scan4, jnp.int32(0))
    cntv[...] = jnp.full((16,), cnt, jnp.int32)
    pltpu.sync_copy(cntv, cnt_o.at[pl.ds(wid * 16, 16)])
    pltpu.sync_copy(cvalv, cval_o.at[pl.ds(wid * CAP, CAP)])
    pltpu.sync_copy(cidxv, cidx_o.at[pl.ds(wid * CAP, CAP)])


@functools.partial(
    pl.kernel,
    mesh=_mesh,
    compiler_params=_params,
    out_type=[
        jax.ShapeDtypeStruct((K,), jnp.float32),
        jax.ShapeDtypeStruct((K,), jnp.int32),
        jax.ShapeDtypeStruct((K,), jnp.int32),
    ],
    scratch_types=[
        pltpu.VMEM((NW * 16,), jnp.int32),
        pltpu.VMEM((NW * CAP,), jnp.float32),
        pltpu.VMEM((NW * CAP,), jnp.int32),
        pltpu.VMEM((NW * CAP + 16,), jnp.float32),
        pltpu.VMEM((NW * CAP + 16,), jnp.int32),
        pltpu.VMEM((K,), jnp.float32),
        pltpu.VMEM((K,), jnp.int32),
        pltpu.VMEM((K,), jnp.int32),
    ],
)
def _stage3(cnt_i, cval_i, cidx_i, s_o, h_o, t_o,
            cntv, cvalv, cidxv, mval, midx, osv, ohv, otv):
    wid = _wid()

    @pl.when(wid == 0)
    def _():
        pltpu.sync_copy(cnt_i, cntv)
        pltpu.sync_copy(cval_i, cvalv)
        pltpu.sync_copy(cidx_i, cidxv)
        neg = _neg16()
        iota = lax.iota(jnp.int32, 16)
        lane0 = iota == 0

        def cw(w, M):
            cnt = cntv[pl.ds(w * 16, 16)][0]
            nv = (cnt + 15) // 16

            def cj(j, z):
                mval[pl.ds(M + j * 16, 16)] = cvalv[pl.ds(w * CAP + j * 16, 16)]
                midx[pl.ds(M + j * 16, 16)] = cidxv[pl.ds(w * CAP + j * 16, 16)]
                return z

            lax.fori_loop(0, nv, cj, jnp.int32(0))
            return M + cnt

        M = lax.fori_loop(0, NW, cw, jnp.int32(0))
        nvv = (M + 15) // 16

        def rnd(k, z):
            def mx(j, mm):
                return jnp.maximum(mm, mval[pl.ds(j * 16, 16)])

            m = _vmax(lax.fori_loop(0, nvv, mx, neg))

            def mi(j, rm):
                v = mval[pl.ds(j * 16, 16)]
                iv = midx[pl.ds(j * 16, 16)]
                return jnp.minimum(rm, jnp.where(v == m, iv, IMAX))

            fi = _vmin(lax.fori_loop(0, nvv, mi, jnp.full((16,), IMAX, jnp.int32)))

            def mo(j, z2):
                v = mval[pl.ds(j * 16, 16)]
                iv = midx[pl.ds(j * 16, 16)]
                mval[pl.ds(j * 16, 16)] = jnp.where(
                    (v == m) & (iv == fi), jnp.float32(-jnp.inf), v)
                return z2

            lax.fori_loop(0, nvv, mo, jnp.int32(0))
            kv = jnp.full((16,), k, jnp.int32)
            hidx = fi // V
            plsc.store_scatter(osv, [kv], jnp.full((16,), m, jnp.float32), mask=lane0)
            plsc.store_scatter(ohv, [kv], jnp.full((16,), hidx, jnp.int32), mask=lane0)
            plsc.store_scatter(otv, [kv], jnp.full((16,), fi - hidx * V, jnp.int32),
                               mask=lane0)
            return z

        lax.fori_loop(0, K, rnd, jnp.int32(0))
        pltpu.sync_copy(osv, s_o)
        pltpu.sync_copy(ohv, h_o)
        pltpu.sync_copy(otv, t_o)


def kernel(hypo_scores, next_token_probs, beam_width):
    del beam_width
    flat = next_token_probs.T.reshape(-1)
    cnts, cval, cidx = _stage12(flat, hypo_scores)
    scores, hidx, tok = _stage3(cnts, cval, cidx)
    return scores, hidx, tok


# speedup vs baseline: 254.5353x; 1.0140x over previous
import functools

import jax
import jax.numpy as jnp
from jax import lax
from jax.experimental import pallas as pl
from jax.experimental.pallas import tpu as pltpu
from jax.experimental.pallas import tpu_sc as plsc

R = 128
V = 100000
N = R * V
NC, NS = 2, 16
NW = NC * NS
BLK = 1024
NB = N // BLK
BPW = 391
CHB = 23
CHW = CHB * BLK
NCH = 18
K = 32
CAP = 512
IMAX = 2147483647

_mesh = plsc.VectorSubcoreMesh(core_axis_name="c", subcore_axis_name="s")
_params = pltpu.CompilerParams(needs_layout_passes=False)


def _wid():
    return lax.axis_index("c") * NS + lax.axis_index("s")


def _neg16():
    return jnp.full((16,), -jnp.inf, jnp.float32)


def _vmax(v):
    out = plsc.sort_key_val(v, v, descending=True)
    k = out[0] if isinstance(out, (tuple, list)) else out
    return k[0]


def _vmin(v):
    out = plsc.sort_key_val(v, v, descending=False)
    k = out[0] if isinstance(out, (tuple, list)) else out
    return k[0]


def _popcnt(mask):
    return plsc.all_reduce_population_count(mask)[0]


@functools.partial(
    pl.kernel,
    mesh=_mesh,
    compiler_params=_params,
    out_type=[
        jax.ShapeDtypeStruct((NW * 16,), jnp.int32),
        jax.ShapeDtypeStruct((NW * CAP,), jnp.float32),
        jax.ShapeDtypeStruct((NW * CAP,), jnp.int32),
    ],
    scratch_types=[
        pltpu.VMEM((CHW,), jnp.float32),
        pltpu.VMEM((CHW,), jnp.float32),
        pltpu.VMEM((BPW * 16,), jnp.float32),
        pltpu.VMEM((R,), jnp.float32),
        pltpu.VMEM((16,), jnp.float32),
        pltpu.VMEM_SHARED((NS * 16,), jnp.float32),
        pltpu.VMEM((NS * 16,), jnp.float32),
        pltpu.VMEM((BLK,), jnp.float32),
        pltpu.VMEM((CAP,), jnp.float32),
        pltpu.VMEM((CAP,), jnp.int32),
        pltpu.VMEM((16,), jnp.int32),
        pltpu.SemaphoreType.DMA,
        pltpu.SemaphoreType.DMA,
    ],
)
def _stage12(flat, hypo, cnt_o, cval_o, cidx_o, buf0, buf1, trigv, hypov, wmv,
             shared, smaxv, sbuf, cvalv, cidxv, cntv, sem0, sem1):
    wid = _wid()
    start_b = wid * BPW
    nb = jnp.minimum(BPW, NB - start_b)
    pltpu.sync_copy(hypo, hypov)
    hvs = [hypov[pl.ds(i * 16, 16)] for i in range(8)]
    neg = _neg16()

    def cstart(ch):
        return jnp.minimum(ch * CHB, nb - CHB)

    def issue(ch, bf, sem):
        pltpu.make_async_copy(
            flat.at[pl.ds((start_b + cstart(ch)) * BLK, CHW)], bf, sem).start()

    issue(0, buf0, sem0)
    issue(1, buf1, sem1)

    def chunk(ch, bf, wm):
        cs = cstart(ch)

        def blk(bi, wm2):
            o = bi * BLK
            ms = [neg] * 8
            for j in range(64):
                ms[j % 8] = jnp.maximum(ms[j % 8], bf[pl.ds(o + j * 16, 16)])
            t = ms[0] + hvs[0]
            for i in range(1, 8):
                t = jnp.maximum(t, ms[i] + hvs[i])
            trigv[pl.ds((cs + bi) * 16, 16)] = t
            return jnp.maximum(wm2, t)

        return lax.fori_loop(0, CHB, blk, wm)

    def outer(g, wm):
        c0 = 2 * g
        pltpu.make_async_copy(flat.at[pl.ds(0, CHW)], buf0, sem0).wait()
        wm = chunk(c0, buf0, wm)

        @pl.when(c0 + 2 < NCH)
        def _():
            issue(c0 + 2, buf0, sem0)

        pltpu.make_async_copy(flat.at[pl.ds(0, CHW)], buf1, sem1).wait()
        wm = chunk(c0 + 1, buf1, wm)

        @pl.when(c0 + 3 < NCH)
        def _():
            issue(c0 + 3, buf1, sem1)

        return wm

    wm = lax.fori_loop(0, NCH // 2, outer, neg)

    sid = lax.axis_index("s")
    wmv[...] = wm
    pltpu.sync_copy(wmv, shared.at[pl.ds(sid * 16, 16)])
    plsc.subcore_barrier()
    pltpu.sync_copy(shared, smaxv)

    def rnd(k, _):
        mm = neg
        for j in range(NS):
            mm = jnp.maximum(mm, smaxv[pl.ds(j * 16, 16)])
        m = _vmax(mm)
        ninf = jnp.float32(-jnp.inf)
        for j in range(NS):
            v = smaxv[pl.ds(j * 16, 16)]
            smaxv[pl.ds(j * 16, 16)] = jnp.where(v == m, ninf, v)
        return m

    T = lax.fori_loop(0, K, rnd, jnp.float32(0.0))
    Tv = jnp.full((16,), T, jnp.float32)
    iota = lax.iota(jnp.int32, 16)
    iotaV = iota * V

    def ini(j, z):
        cvalv[pl.ds(j * 16, 16)] = neg
        return z

    lax.fori_loop(0, CAP // 16, ini, jnp.int32(0))

    def fetch_emit(blk_abs, cnt3):
        pltpu.sync_copy(flat.at[pl.ds(blk_abs * BLK, BLK)], sbuf)
        c3 = cnt3
        for j in range(64):
            x = sbuf[pl.ds(j * 16, 16)]
            sv = x + hvs[j % 8]
            mk = sv >= Tv
            pc = _popcnt(mk)
            scal = (j % 8) * 16 * V + blk_abs * 8 + j // 8

            def emit(c4, sv=sv, mk=mk, scal=scal):
                c5 = jnp.minimum(c4, CAP - 16)
                plsc.store_compressed(cvalv.at[pl.ds(c5, 16)], sv, mask=mk)
                plsc.store_compressed(
                    cidxv.at[pl.ds(c5, 16)], iotaV + scal, mask=mk)
                return c5 + pc

            c3 = lax.cond(pc > 0, emit, lambda c4: c4, c3)
        return c3

    nbc = jnp.maximum(nb - 1, 0)

    def scan_blk(bi, cnt2):
        tvec = trigv[pl.ds(jnp.minimum(bi, nbc) * 16, 16)]
        hit = (_popcnt(tvec >= Tv) > 0) & (bi < nb)
        return lax.cond(
            hit, lambda c3: fetch_emit(start_b + bi, c3), lambda c3: c3, cnt2)

    def scan4(g, cnt):
        b0 = g * 4
        t = trigv[pl.ds(jnp.minimum(b0, nbc) * 16, 16)]
        for u in range(1, 4):
            t = jnp.maximum(t, trigv[pl.ds(jnp.minimum(b0 + u, nbc) * 16, 16)])

        def slow(cnt2):
            return lax.fori_loop(b0, b0 + 4, scan_blk, cnt2)

        return lax.cond(_popcnt(t >= Tv) > 0, slow, lambda c: c, cnt)

    cnt = lax.fori_loop(0, (BPW + 3) // 4, scan4, jnp.int32(0))
    cntv[...] = jnp.full((16,), cnt, jnp.int32)
    pltpu.sync_copy(cntv, cnt_o.at[pl.ds(wid * 16, 16)])
    pltpu.sync_copy(cvalv, cval_o.at[pl.ds(wid * CAP, CAP)])
    pltpu.sync_copy(cidxv, cidx_o.at[pl.ds(wid * CAP, CAP)])


@functools.partial(
    pl.kernel,
    mesh=_mesh,
    compiler_params=_params,
    out_type=[
        jax.ShapeDtypeStruct((K,), jnp.float32),
        jax.ShapeDtypeStruct((K,), jnp.int32),
        jax.ShapeDtypeStruct((K,), jnp.int32),
    ],
    scratch_types=[
        pltpu.VMEM((NW * 16,), jnp.int32),
        pltpu.VMEM((NW * CAP,), jnp.float32),
        pltpu.VMEM((NW * CAP,), jnp.int32),
        pltpu.VMEM((NW * CAP + 16,), jnp.float32),
        pltpu.VMEM((NW * CAP + 16,), jnp.int32),
        pltpu.VMEM((K,), jnp.float32),
        pltpu.VMEM((K,), jnp.int32),
        pltpu.VMEM((K,), jnp.int32),
        pltpu.SemaphoreType.DMA,
    ],
)
def _stage3(cnt_i, cval_i, cidx_i, s_o, h_o, t_o,
            cntv, cvalv, cidxv, mval, midx, osv, ohv, otv, sem):
    wid = _wid()

    @pl.when(wid == 0)
    def _():
        d0 = pltpu.make_async_copy(cnt_i, cntv, sem)
        d1 = pltpu.make_async_copy(cval_i, cvalv, sem)
        d2 = pltpu.make_async_copy(cidx_i, cidxv, sem)
        d0.start(); d1.start(); d2.start()
        d0.wait(); d1.wait(); d2.wait()
        neg = _neg16()
        iota = lax.iota(jnp.int32, 16)
        lane0 = iota == 0

        def cw(w, M):
            cnt = cntv[pl.ds(w * 16, 16)][0]
            nv = (cnt + 15) // 16

            def cj(j, z):
                mval[pl.ds(M + j * 16, 16)] = cvalv[pl.ds(w * CAP + j * 16, 16)]
                midx[pl.ds(M + j * 16, 16)] = cidxv[pl.ds(w * CAP + j * 16, 16)]
                return z

            lax.fori_loop(0, nv, cj, jnp.int32(0))
            return M + cnt

        M = lax.fori_loop(0, NW, cw, jnp.int32(0))
        nvv = (M + 15) // 16

        def rnd(k, z):
            def mx(j, mm):
                return jnp.maximum(mm, mval[pl.ds(j * 16, 16)])

            m = _vmax(lax.fori_loop(0, nvv, mx, neg))

            def mi(j, rm):
                v = mval[pl.ds(j * 16, 16)]
                iv = midx[pl.ds(j * 16, 16)]
                return jnp.minimum(rm, jnp.where(v == m, iv, IMAX))

            fi = _vmin(lax.fori_loop(0, nvv, mi, jnp.full((16,), IMAX, jnp.int32)))

            def mo(j, z2):
                v = mval[pl.ds(j * 16, 16)]
                iv = midx[pl.ds(j * 16, 16)]
                mval[pl.ds(j * 16, 16)] = jnp.where(
                    (v == m) & (iv == fi), jnp.float32(-jnp.inf), v)
                return z2

            lax.fori_loop(0, nvv, mo, jnp.int32(0))
            kv = jnp.full((16,), k, jnp.int32)
            hidx = fi // V
            plsc.store_scatter(osv, [kv], jnp.full((16,), m, jnp.float32), mask=lane0)
            plsc.store_scatter(ohv, [kv], jnp.full((16,), hidx, jnp.int32), mask=lane0)
            plsc.store_scatter(otv, [kv], jnp.full((16,), fi - hidx * V, jnp.int32),
                               mask=lane0)
            return z

        lax.fori_loop(0, K, rnd, jnp.int32(0))
        o0 = pltpu.make_async_copy(osv, s_o, sem)
        o1 = pltpu.make_async_copy(ohv, h_o, sem)
        o2 = pltpu.make_async_copy(otv, t_o, sem)
        o0.start(); o1.start(); o2.start()
        o0.wait(); o1.wait(); o2.wait()


def kernel(hypo_scores, next_token_probs, beam_width):
    del beam_width
    flat = next_token_probs.T.reshape(-1)
    cnts, cval, cidx = _stage12(flat, hypo_scores)
    scores, hidx, tok = _stage3(cnts, cval, cidx)
    return scores, hidx, tok
